# baseline (speedup 1.0000x reference)
import jax
import jax.numpy as jnp
from jax.experimental import pallas as pl
from jax.experimental.pallas import tpu as pltpu

B, N, DIN, DH = 128, 64, 128, 32
BB = 16
R = BB * N


def _lrelu(v):
    return jnp.maximum(v, 0.2 * v)


def _exp_rows(e):
    m = jnp.max(e, axis=-1, keepdims=True)
    ex = jnp.exp(e - m)
    return ex, jnp.sum(ex, axis=-1, keepdims=True)


def _gat2_body(x_ref, w1_ref, as1_ref, ad1_ref, b1_ref, gamma_ref, beta_ref,
               w2_ref, sc2_ref, sel_ref, out_ref):
    xb = x_ref[...].reshape(R, DIN)
    sel = sel_ref[...]

    h = jnp.dot(xb, w1_ref[...], preferred_element_type=jnp.float32)
    h3 = h.reshape(BB, N, DH)
    as_s = jnp.sum(h3 * as1_ref[...][None], axis=-1)
    ad_c = jnp.dot(h, ad1_ref[...].T, preferred_element_type=jnp.float32)
    t_as = jnp.dot(sel, as_s, preferred_element_type=jnp.float32)
    ex, den = _exp_rows(_lrelu(t_as + ad_c))
    agg = jnp.concatenate(
        [jnp.dot(ex[b * N:(b + 1) * N], h3[b],
                 preferred_element_type=jnp.float32) for b in range(BB)],
        axis=0)
    out1 = agg * (1.0 / den) + b1_ref[...]

    mu = jnp.mean(out1, axis=-1, keepdims=True)
    var = jnp.mean((out1 - mu) ** 2, axis=-1, keepdims=True)
    hn = (out1 - mu) * jax.lax.rsqrt(var + 1e-5) * gamma_ref[...] + beta_ref[...]
    ha = jnp.where(hn > 0, hn, jnp.exp(jnp.minimum(hn, 0.0)) - 1.0)

    g = jnp.dot(ha, w2_ref[...].T, preferred_element_type=jnp.float32)
    g_s = jnp.sum(ha.reshape(BB, N, DH) * w2_ref[...][None], axis=-1)
    a_s2 = sc2_ref[0, 0]
    a_d2 = sc2_ref[0, 1]
    b2 = sc2_ref[0, 2]
    g_t = jnp.dot(sel, g_s, preferred_element_type=jnp.float32)
    ex2, den2 = _exp_rows(_lrelu(a_s2 * g_t + a_d2 * g))
    num2 = jnp.sum(ex2 * g_t, axis=-1, keepdims=True)
    out_ref[...] = num2 * (1.0 / den2) + b2


def kernel(x, adj, W1, att_src1, att_dst1, b1, gamma, beta, W2, att_src2,
           att_dst2, b2):
    del adj
    as1 = att_src1.reshape(1, DH)
    ad1 = att_dst1.reshape(1, DH)
    b1r = b1.reshape(1, DH)
    g1 = gamma.reshape(1, DH)
    be1 = beta.reshape(1, DH)
    w2r = W2.reshape(1, DH)
    sc2 = jnp.stack([att_src2.reshape(()), att_dst2.reshape(()),
                     b2.reshape(())]).reshape(1, 3)
    sel = (jnp.arange(R, dtype=jnp.int32)[:, None] // N ==
           jnp.arange(BB, dtype=jnp.int32)[None, :]).astype(jnp.float32)

    full = lambda shape: pl.BlockSpec(shape, lambda i: (0,) * len(shape))
    out = pl.pallas_call(
        _gat2_body,
        grid=(B // BB,),
        in_specs=[
            pl.BlockSpec((BB, N, DIN), lambda i: (i, 0, 0)),
            full((DIN, DH)),
            full((1, DH)), full((1, DH)), full((1, DH)),
            full((1, DH)), full((1, DH)), full((1, DH)),
            full((1, 3)),
            full((R, BB)),
        ],
        out_specs=pl.BlockSpec((R, 1), lambda i: (i, 0)),
        out_shape=jax.ShapeDtypeStruct((B * N, 1), jnp.float32),
        compiler_params=pltpu.CompilerParams(
            dimension_semantics=("parallel",)),
    )(x, W1, as1, ad1, b1r, g1, be1, w2r, sc2, sel)
    return out.reshape(B, N, 1)


# speedup vs baseline: 1.0764x; 1.0764x over previous
import jax
import jax.numpy as jnp
from jax.experimental import pallas as pl
from jax.experimental.pallas import tpu as pltpu

B, N, DIN, DH = 128, 64, 128, 32
BB = 32
R = BB * N


def _lrelu(v):
    return jnp.maximum(v, 0.2 * v)


def _exp_rows(e):
    m = jnp.max(e, axis=-1, keepdims=True)
    ex = jnp.exp(e - m)
    return ex, jnp.sum(ex, axis=-1, keepdims=True)


def _gat2_body(x_ref, w1_ref, as1_ref, ad1_ref, b1_ref, gamma_ref, beta_ref,
               w2_ref, sc2_ref, sel_ref, out_ref):
    xb = x_ref[...].reshape(R, DIN)
    sel = sel_ref[...]

    h = jnp.dot(xb, w1_ref[...], preferred_element_type=jnp.float32)
    h3 = h.reshape(BB, N, DH)
    as_s = jnp.sum(h3 * as1_ref[...][None], axis=-1)
    ad_c = jnp.dot(h, ad1_ref[...].T, preferred_element_type=jnp.float32)
    t_as = jnp.dot(sel, as_s, preferred_element_type=jnp.float32)
    ex, den = _exp_rows(_lrelu(t_as + ad_c))
    agg = jnp.concatenate(
        [jnp.dot(ex[b * N:(b + 1) * N], h3[b],
                 preferred_element_type=jnp.float32) for b in range(BB)],
        axis=0)
    out1 = agg * (1.0 / den) + b1_ref[...]

    mu = jnp.mean(out1, axis=-1, keepdims=True)
    var = jnp.mean((out1 - mu) ** 2, axis=-1, keepdims=True)
    hn = (out1 - mu) * jax.lax.rsqrt(var + 1e-5) * gamma_ref[...] + beta_ref[...]
    ha = jnp.where(hn > 0, hn, jnp.exp(jnp.minimum(hn, 0.0)) - 1.0)

    g = jnp.dot(ha, w2_ref[...].T, preferred_element_type=jnp.float32)
    g_s = jnp.sum(ha.reshape(BB, N, DH) * w2_ref[...][None], axis=-1)
    a_s2 = sc2_ref[0, 0]
    a_d2 = sc2_ref[0, 1]
    b2 = sc2_ref[0, 2]
    g_t = jnp.dot(sel, g_s, preferred_element_type=jnp.float32)
    ex2, den2 = _exp_rows(_lrelu(a_s2 * g_t + a_d2 * g))
    num2 = jnp.sum(ex2 * g_t, axis=-1, keepdims=True)
    out_ref[...] = num2 * (1.0 / den2) + b2


def kernel(x, adj, W1, att_src1, att_dst1, b1, gamma, beta, W2, att_src2,
           att_dst2, b2):
    del adj
    as1 = att_src1.reshape(1, DH)
    ad1 = att_dst1.reshape(1, DH)
    b1r = b1.reshape(1, DH)
    g1 = gamma.reshape(1, DH)
    be1 = beta.reshape(1, DH)
    w2r = W2.reshape(1, DH)
    sc2 = jnp.stack([att_src2.reshape(()), att_dst2.reshape(()),
                     b2.reshape(())]).reshape(1, 3)
    sel = (jnp.arange(R, dtype=jnp.int32)[:, None] // N ==
           jnp.arange(BB, dtype=jnp.int32)[None, :]).astype(jnp.float32)

    full = lambda shape: pl.BlockSpec(shape, lambda i: (0,) * len(shape))
    out = pl.pallas_call(
        _gat2_body,
        grid=(B // BB,),
        in_specs=[
            pl.BlockSpec((BB, N, DIN), lambda i: (i, 0, 0)),
            full((DIN, DH)),
            full((1, DH)), full((1, DH)), full((1, DH)),
            full((1, DH)), full((1, DH)), full((1, DH)),
            full((1, 3)),
            full((R, BB)),
        ],
        out_specs=pl.BlockSpec((R, 1), lambda i: (i, 0)),
        out_shape=jax.ShapeDtypeStruct((B * N, 1), jnp.float32),
        compiler_params=pltpu.CompilerParams(
            dimension_semantics=("parallel",)),
    )(x, W1, as1, ad1, b1r, g1, be1, w2r, sc2, sel)
    return out.reshape(B, N, 1)


# speedup vs baseline: 1.0914x; 1.0139x over previous
import jax
import jax.numpy as jnp
import numpy as np
from jax.experimental import pallas as pl
from jax.experimental.pallas import tpu as pltpu

B, N, DIN, DH = 128, 64, 128, 32
BB = 32
R = BB * N

_SEL = np.asarray(
    (np.arange(R)[:, None] // N) == np.arange(BB)[None, :], np.float32)


def _lrelu(v):
    return jnp.maximum(v, 0.2 * v)


def _exp_rows(e):
    m = jnp.max(e, axis=-1, keepdims=True)
    ex = jnp.exp(e - m)
    return ex, jnp.sum(ex, axis=-1, keepdims=True)


def _gat2_body(x_ref, w1_ref, as1_ref, ad1_ref, b1_ref, gamma_ref, beta_ref,
               w2_ref, as2_ref, ad2_ref, b2_ref, sel_ref, out_ref):
    xb = x_ref[...].reshape(R, DIN)
    sel = sel_ref[...]

    h = jnp.dot(xb, w1_ref[...], preferred_element_type=jnp.float32)
    h3 = h.reshape(BB, N, DH)
    as_s = jnp.sum(h3 * as1_ref[...][None], axis=-1)
    ad_c = jnp.dot(h, ad1_ref[...].T, preferred_element_type=jnp.float32)
    t_as = jnp.dot(sel, as_s, preferred_element_type=jnp.float32)
    ex, den = _exp_rows(_lrelu(t_as + ad_c))
    agg = jnp.concatenate(
        [jnp.dot(ex[b * N:(b + 1) * N], h3[b],
                 preferred_element_type=jnp.float32) for b in range(BB)],
        axis=0)
    out1 = agg * (1.0 / den) + b1_ref[...]

    mu = jnp.mean(out1, axis=-1, keepdims=True)
    var = jnp.mean((out1 - mu) ** 2, axis=-1, keepdims=True)
    hn = (out1 - mu) * jax.lax.rsqrt(var + 1e-5) * gamma_ref[...] + beta_ref[...]
    ha = jnp.where(hn > 0, hn, jnp.exp(jnp.minimum(hn, 0.0)) - 1.0)

    w2_row = w2_ref[...].T
    g = jnp.dot(ha, w2_ref[...], preferred_element_type=jnp.float32)
    g_s = jnp.sum(ha.reshape(BB, N, DH) * w2_row[None], axis=-1)
    a_s2 = as2_ref[0, 0]
    a_d2 = ad2_ref[0, 0]
    g_t = jnp.dot(sel, g_s, preferred_element_type=jnp.float32)
    ex2, den2 = _exp_rows(_lrelu(a_s2 * g_t + a_d2 * g))
    num2 = jnp.sum(ex2 * g_t, axis=-1, keepdims=True)
    out_ref[...] = num2 * (1.0 / den2) + b2_ref[0, 0]


def kernel(x, adj, W1, att_src1, att_dst1, b1, gamma, beta, W2, att_src2,
           att_dst2, b2):
    del adj
    as1 = att_src1.reshape(1, DH)
    ad1 = att_dst1.reshape(1, DH)
    b1r = b1.reshape(1, DH)
    g1 = gamma.reshape(1, DH)
    be1 = beta.reshape(1, DH)
    as2 = att_src2.reshape(1, 1)
    ad2 = att_dst2.reshape(1, 1)
    b2r = b2.reshape(1, 1)
    sel = jnp.asarray(_SEL)

    full = lambda shape: pl.BlockSpec(shape, lambda i: (0,) * len(shape))
    out = pl.pallas_call(
        _gat2_body,
        grid=(B // BB,),
        in_specs=[
            pl.BlockSpec((BB, N, DIN), lambda i: (i, 0, 0)),
            full((DIN, DH)),
            full((1, DH)), full((1, DH)), full((1, DH)),
            full((1, DH)), full((1, DH)),
            full((DH, 1)),
            full((1, 1)), full((1, 1)), full((1, 1)),
            full((R, BB)),
        ],
        out_specs=pl.BlockSpec((R, 1), lambda i: (i, 0)),
        out_shape=jax.ShapeDtypeStruct((B * N, 1), jnp.float32),
        compiler_params=pltpu.CompilerParams(
            dimension_semantics=("parallel",)),
    )(x, W1, as1, ad1, b1r, g1, be1, W2, as2, ad2, b2r, sel)
    return out.reshape(B, N, 1)


# speedup vs baseline: 1.2007x; 1.1002x over previous
import jax
import jax.numpy as jnp
import numpy as np
from jax.experimental import pallas as pl
from jax.experimental.pallas import tpu as pltpu

B, N, DIN, DH = 128, 64, 128, 32
BB = 128
R = BB * N

_SEL = np.asarray(
    (np.arange(R)[:, None] // N) == np.arange(BB)[None, :], np.float32)


def _lrelu(v):
    return jnp.maximum(v, 0.2 * v)


def _exp_rows(e):
    m = jnp.max(e, axis=-1, keepdims=True)
    ex = jnp.exp(e - m)
    return ex, jnp.sum(ex, axis=-1, keepdims=True)


def _per_slate_rows(col):
    return jnp.swapaxes(col.reshape(BB, N, 1), 1, 2).reshape(BB, N)


def _gat2_body(x_ref, w1_ref, as1_ref, ad1_ref, b1_ref, gamma_ref, beta_ref,
               w2_ref, as2_ref, ad2_ref, b2_ref, sel_ref, out_ref):
    xb = x_ref[...].reshape(R, DIN)
    sel = sel_ref[...]

    h = jnp.dot(xb, w1_ref[...], preferred_element_type=jnp.float32)
    h3 = h.reshape(BB, N, DH)
    as_s = jnp.sum(h3 * as1_ref[...][None], axis=-1)
    ad_c = jnp.dot(h, ad1_ref[...].T, preferred_element_type=jnp.float32)
    t_as = jnp.dot(sel, as_s, preferred_element_type=jnp.float32)
    ex, den = _exp_rows(_lrelu(t_as + ad_c))
    agg = jnp.concatenate(
        [jnp.dot(ex[b * N:(b + 1) * N], h3[b],
                 preferred_element_type=jnp.float32) for b in range(BB)],
        axis=0)
    out1 = agg * (1.0 / den) + b1_ref[...]

    mu = jnp.mean(out1, axis=-1, keepdims=True)
    var = jnp.mean((out1 - mu) ** 2, axis=-1, keepdims=True)
    hn = (out1 - mu) * jax.lax.rsqrt(var + 1e-5) * gamma_ref[...] + beta_ref[...]
    ha = jnp.where(hn > 0, hn, jnp.exp(jnp.minimum(hn, 0.0)) - 1.0)

    w2_row = w2_ref[...].T
    g = jnp.dot(ha, w2_ref[...], preferred_element_type=jnp.float32)
    g_s = jnp.sum(ha.reshape(BB, N, DH) * w2_row[None], axis=-1)
    a_s2 = as2_ref[0, 0]
    a_d2 = ad2_ref[0, 0]
    g_t = jnp.dot(sel, g_s, preferred_element_type=jnp.float32)
    ex2, den2 = _exp_rows(_lrelu(a_s2 * g_t + a_d2 * g))
    num2 = jnp.sum(ex2 * g_t, axis=-1, keepdims=True)
    res = num2 * (1.0 / den2) + b2_ref[0, 0]
    out_ref[...] = _per_slate_rows(res).T


def kernel(x, adj, W1, att_src1, att_dst1, b1, gamma, beta, W2, att_src2,
           att_dst2, b2):
    del adj
    as1 = att_src1.reshape(1, DH)
    ad1 = att_dst1.reshape(1, DH)
    b1r = b1.reshape(1, DH)
    g1 = gamma.reshape(1, DH)
    be1 = beta.reshape(1, DH)
    as2 = att_src2.reshape(1, 1)
    ad2 = att_dst2.reshape(1, 1)
    b2r = b2.reshape(1, 1)
    sel = jnp.asarray(_SEL)

    full = lambda shape: pl.BlockSpec(shape, lambda i: (0,) * len(shape))
    out_nb = pl.pallas_call(
        _gat2_body,
        grid=(B // BB,),
        in_specs=[
            pl.BlockSpec((BB, N, DIN), lambda i: (i, 0, 0)),
            full((DIN, DH)),
            full((1, DH)), full((1, DH)), full((1, DH)),
            full((1, DH)), full((1, DH)),
            full((DH, 1)),
            full((1, 1)), full((1, 1)), full((1, 1)),
            full((R, BB)),
        ],
        out_specs=pl.BlockSpec((N, BB), lambda i: (0, i)),
        out_shape=jax.ShapeDtypeStruct((N, B), jnp.float32),
    )(x, W1, as1, ad1, b1r, g1, be1, W2, as2, ad2, b2r, sel)
    return out_nb.T.reshape(B, N, 1)


# speedup vs baseline: 1.2445x; 1.0365x over previous
import jax
import jax.numpy as jnp
import numpy as np
from jax.experimental import pallas as pl
from jax.experimental.pallas import tpu as pltpu

B, N, DIN, DH = 128, 64, 128, 32
BB = 64
R = BB * N

_SEL = np.asarray(
    (np.arange(R)[:, None] // N) == np.arange(BB)[None, :], np.float32)


def _lrelu(v):
    return jnp.maximum(v, 0.2 * v)


def _exp_rows(e):
    m = jnp.max(e, axis=-1, keepdims=True)
    ex = jnp.exp(e - m)
    return ex, jnp.sum(ex, axis=-1, keepdims=True)


def _per_slate_rows(col):
    return jnp.swapaxes(col.reshape(BB, N, 1), 1, 2).reshape(BB, N)


def _gat2_body(x_ref, w1_ref, as1_ref, ad1_ref, b1_ref, gamma_ref, beta_ref,
               w2_ref, as2_ref, ad2_ref, b2_ref, sel_ref, out_ref):
    xb = x_ref[...].reshape(R, DIN)
    sel = sel_ref[...]

    h = jnp.dot(xb, w1_ref[...], preferred_element_type=jnp.float32)
    h3 = h.reshape(BB, N, DH)
    as_s = jnp.sum(h3 * as1_ref[...][None], axis=-1)
    ad_c = jnp.dot(h, ad1_ref[...].T, preferred_element_type=jnp.float32)
    t_as = jnp.dot(sel, as_s, preferred_element_type=jnp.float32)
    ex, den = _exp_rows(_lrelu(t_as + ad_c))
    agg = jnp.concatenate(
        [jnp.dot(ex[b * N:(b + 1) * N], h3[b],
                 preferred_element_type=jnp.float32) for b in range(BB)],
        axis=0)
    out1 = agg * (1.0 / den) + b1_ref[...]

    mu = jnp.mean(out1, axis=-1, keepdims=True)
    var = jnp.mean((out1 - mu) ** 2, axis=-1, keepdims=True)
    hn = (out1 - mu) * jax.lax.rsqrt(var + 1e-5) * gamma_ref[...] + beta_ref[...]
    ha = jnp.where(hn > 0, hn, jnp.exp(jnp.minimum(hn, 0.0)) - 1.0)

    w2_row = w2_ref[...].T
    g = jnp.dot(ha, w2_ref[...], preferred_element_type=jnp.float32)
    g_s = jnp.sum(ha.reshape(BB, N, DH) * w2_row[None], axis=-1)
    a_s2 = as2_ref[0, 0]
    a_d2 = ad2_ref[0, 0]
    g_t = jnp.dot(sel, g_s, preferred_element_type=jnp.float32)
    ex2, den2 = _exp_rows(_lrelu(a_s2 * g_t + a_d2 * g))
    num2 = jnp.sum(ex2 * g_t, axis=-1, keepdims=True)
    res = num2 * (1.0 / den2) + b2_ref[0, 0]
    mine = _per_slate_rows(res).T
    mine2 = jnp.concatenate([mine, mine], axis=1)
    lane_blk = jax.lax.broadcasted_iota(jnp.int32, (N, B), 1) // BB
    out_ref[...] = jnp.where(lane_blk == pl.program_id(0), mine2,
                             out_ref[...])


def kernel(x, adj, W1, att_src1, att_dst1, b1, gamma, beta, W2, att_src2,
           att_dst2, b2):
    del adj
    as1 = att_src1.reshape(1, DH)
    ad1 = att_dst1.reshape(1, DH)
    b1r = b1.reshape(1, DH)
    g1 = gamma.reshape(1, DH)
    be1 = beta.reshape(1, DH)
    as2 = att_src2.reshape(1, 1)
    ad2 = att_dst2.reshape(1, 1)
    b2r = b2.reshape(1, 1)
    sel = jnp.asarray(_SEL)

    full = lambda shape: pl.BlockSpec(shape, lambda i: (0,) * len(shape))
    out_nb = pl.pallas_call(
        _gat2_body,
        grid=(B // BB,),
        in_specs=[
            pl.BlockSpec((BB, N, DIN), lambda i: (i, 0, 0)),
            full((DIN, DH)),
            full((1, DH)), full((1, DH)), full((1, DH)),
            full((1, DH)), full((1, DH)),
            full((DH, 1)),
            full((1, 1)), full((1, 1)), full((1, 1)),
            full((R, BB)),
        ],
        out_specs=pl.BlockSpec((N, B), lambda i: (0, 0)),
        out_shape=jax.ShapeDtypeStruct((N, B), jnp.float32),
    )(x, W1, as1, ad1, b1r, g1, be1, W2, as2, ad2, b2r, sel)
    return out_nb.T.reshape(B, N, 1)
